# baseline (speedup 1.0000x reference)
import numpy as np
import jax
import jax.numpy as jnp
from jax.experimental import pallas as pl
from jax.experimental.pallas import tpu as pltpu

_TB = 128


def _band_mask(ndj, w_in, half):
    m = np.zeros((ndj, w_in, 2, half), np.float32)
    for dj in range(ndj):
        for jp in range(2):
            for jj in range(half):
                m[dj, 2 * jj + jp + dj, jp, jj] = 1.0
    return m


def _band_mask_s2(ndj, win, zblk):
    m = np.zeros((ndj, win, 2, zblk), np.float32)
    for dj in range(ndj):
        for zl in range(zblk):
            j = 2 * zl + dj
            jw, jp = j // 2, j % 2
            if jw < win:
                m[dj, jw, jp, zl] = 1.0
    return m


_M1I = _band_mask(5, 28, 12)
_M2I = _band_mask(5, 12, 4)
_M1A = _band_mask(3, 64, 31)
_M2A = _band_mask_s2(3, 7, 6)


def _fused_body(img_ref, aud_ref,
                b1i_ref, ib1_ref, is1_ref, it1_ref,
                b2i_ref, ib2_ref, is2_ref, it2_ref,
                ifw_ref, ifb_ref,
                b1a_ref, ind_ref,
                b2a_ref,
                afw_ref, afb_ref,
                w1i_ref, w1a_ref, fb1_ref, w2f_ref, fb2_ref,
                out_ref):
    tb = img_ref.shape[1]
    f32 = jnp.float32
    bf16 = jnp.bfloat16

    x = img_ref[...].astype(bf16)
    xc = jnp.concatenate([x[di:di + 24] for di in range(5)], axis=-1)
    h = jnp.dot(xc.reshape(24 * tb, 140), b1i_ref[...],
                preferred_element_type=f32)
    h = h.reshape(12, 2, tb, 768)
    h = jnp.max(h, axis=1)
    h = jnp.maximum(h[:, :, :384], h[:, :, 384:])
    h = jnp.maximum(h + ib1_ref[...].reshape(1, 1, 384), 0.0)
    h = h * is1_ref[...].reshape(1, 1, 384) + it1_ref[...].reshape(1, 1, 384)

    x2 = h.astype(bf16)
    x2c = jnp.concatenate([x2[di:di + 8] for di in range(5)], axis=-1)
    h2 = jnp.dot(x2c.reshape(8 * tb, 1920), b2i_ref[...],
                 preferred_element_type=f32)
    h2 = h2.reshape(4, 2, tb, 512)
    h2 = jnp.max(h2, axis=1)
    h2 = jnp.maximum(h2[:, :, :256], h2[:, :, 256:])
    h2 = jnp.maximum(h2 + ib2_ref[...].reshape(1, 1, 256), 0.0)
    h2 = h2 * is2_ref[...].reshape(1, 1, 256) + it2_ref[...].reshape(1, 1, 256)
    feat = h2.astype(bf16)

    img_repr = ifb_ref[...]
    for y in range(4):
        img_repr = img_repr + jnp.dot(feat[y], ifw_ref[y],
                                      preferred_element_type=f32)

    a = aud_ref[...].astype(bf16)
    ind1 = jnp.broadcast_to(ind_ref[...].reshape(1, 1, 64), (26, tb, 64))
    ac = jnp.concatenate([a[di:di + 26] for di in range(3)] + [ind1], axis=-1)
    ha = jnp.dot(ac.reshape(26 * tb, 256), b1a_ref[...],
                 preferred_element_type=f32)
    a1 = jnp.maximum(ha.astype(bf16), 0).reshape(13, 2, tb, 1984)

    ev, od = a1[:, 0], a1[:, 1]
    ind2 = jnp.broadcast_to(ind_ref[...].reshape(1, 1, 64), (12, tb, 64))
    blocks = []
    for z0 in (0, 6, 12, 18, 24):
        sl = slice(z0 * 64, z0 * 64 + 448)
        acat = jnp.concatenate(
            [ev[0:12, :, sl], od[0:12, :, sl], ev[1:13, :, sl], ind2], axis=-1)
        blocks.append(jnp.dot(acat.reshape(12 * tb, 1408), b2a_ref[...],
                              preferred_element_type=f32))
    h2a = jnp.concatenate(blocks, axis=-1)
    a2 = jnp.maximum(h2a.astype(bf16), 0).reshape(12, tb, 960)

    aud_repr = afb_ref[...]
    for y in range(12):
        aud_repr = aud_repr + jnp.dot(a2[y], afw_ref[y],
                                      preferred_element_type=f32)

    hf = (jnp.dot(jnp.maximum(img_repr, 0.0), w1i_ref[...],
                  preferred_element_type=f32)
          + jnp.dot(jnp.maximum(aud_repr, 0.0), w1a_ref[...],
                    preferred_element_type=f32)
          + fb1_ref[...])
    hf = jnp.maximum(hf, 0.0)
    out = jnp.dot(hf, w2f_ref[...], preferred_element_type=f32) + fb2_ref[...]
    out_ref[...] = out.astype(out_ref.dtype)


def kernel(image, audio, img_w1_hwio, img_b1, img_w1, img_bn1_s, img_bn1_t,
           img_w2_hwio, img_b2, img_w2, img_bn2_s, img_bn2_t, img_fc_w,
           img_fc_b, aud_w1_hwio, aud_b1, aud_w1, aud_w2_hwio, aud_b2,
           aud_w2, aud_fc_w, aud_fc_b, fus_w1, fus_b1, fus_w1_img,
           fus_w1_aud, fus_w2, fus_b2):
    b = image.shape[0]
    tb = _TB if b % _TB == 0 else (8 if b % 8 == 0 else 1)
    bf16 = jnp.bfloat16

    img_t = image.reshape(b, 28, 28).transpose(1, 0, 2)
    aud_t = audio.reshape(b, 28, 64).transpose(1, 0, 2)

    w1s = img_w1_hwio.reshape(5, 5, 32)
    b1i = jnp.einsum('awpj,dac->dwpjc', _M1I, w1s
                     ).reshape(5 * 28, 768).astype(bf16)
    b2i = jnp.einsum('awpj,daio->dwipjo', _M2I, img_w2_hwio
                     ).reshape(5 * 384, 512).astype(bf16)
    aw1s = aud_w1_hwio.reshape(3, 3, 32)
    ab1 = jnp.tile(aud_b1, (1, 62))
    b1a = jnp.concatenate(
        [jnp.einsum('awpj,dac->dwjpc', _M1A, aw1s).reshape(3 * 64, 1984),
         ab1, jnp.zeros((63, 1984), jnp.float32)], axis=0
    ).astype(bf16)
    ab2 = jnp.tile(aud_b2, (1, 6))
    b2a = jnp.concatenate(
        [jnp.einsum('ajpz,daio->djpizo', _M2A, aud_w2_hwio).reshape(3 * 448, 192),
         ab2, jnp.zeros((63, 192), jnp.float32)], axis=0
    ).astype(bf16)
    ind = jnp.zeros((1, 64), jnp.float32).at[0, 0].set(1.0).astype(bf16)

    ib1 = jnp.tile(img_b1, (1, 12))
    is1 = jnp.tile(img_bn1_s, (1, 12))
    it1 = jnp.tile(img_bn1_t, (1, 12))
    ib2 = jnp.tile(img_b2, (1, 4))
    is2 = jnp.tile(img_bn2_s, (1, 4))
    it2 = jnp.tile(img_bn2_t, (1, 4))
    ifw = img_fc_w.reshape(4, 256, 64).astype(bf16)
    afw = aud_fc_w.reshape(12, 960, 64).astype(bf16)

    def full(a):
        return pl.BlockSpec(a.shape, lambda i: (0,) * a.ndim)

    out = pl.pallas_call(
        _fused_body,
        out_shape=jax.ShapeDtypeStruct((b, 64), jnp.float32),
        grid=(b // tb,),
        in_specs=[
            pl.BlockSpec((28, tb, 28), lambda i: (0, i, 0)),
            pl.BlockSpec((28, tb, 64), lambda i: (0, i, 0)),
            full(b1i), full(ib1), full(is1), full(it1),
            full(b2i), full(ib2), full(is2), full(it2),
            full(ifw), full(img_fc_b),
            full(b1a), full(ind),
            full(b2a),
            full(afw), full(aud_fc_b),
            full(fus_w1_img), full(fus_w1_aud), full(fus_b1),
            full(fus_w2), full(fus_b2),
        ],
        out_specs=pl.BlockSpec((tb, 64), lambda i: (i, 0)),
        compiler_params=pltpu.CompilerParams(dimension_semantics=("parallel",)),
        cost_estimate=pl.CostEstimate(flops=2 * b * 16_000_000,
                                      transcendentals=0,
                                      bytes_accessed=4 * b * (784 + 1792 + 64)),
    )(img_t, aud_t,
      b1i, ib1, is1, it1,
      b2i, ib2, is2, it2,
      ifw, img_fc_b,
      b1a, ind,
      b2a,
      afw, aud_fc_b,
      fus_w1_img, fus_w1_aud, fus_b1, fus_w2, fus_b2)
    return out


# speedup vs baseline: 1.0057x; 1.0057x over previous
import numpy as np
import jax
import jax.numpy as jnp
from jax.experimental import pallas as pl
from jax.experimental.pallas import tpu as pltpu

_TB = 128


def _band_mask(ndj, w_in, half):
    m = np.zeros((ndj, w_in, 2, half), np.float32)
    for dj in range(ndj):
        for jp in range(2):
            for jj in range(half):
                m[dj, 2 * jj + jp + dj, jp, jj] = 1.0
    return m


def _band_mask_s2(ndj, win, zblk):
    m = np.zeros((ndj, win, 2, zblk), np.float32)
    for dj in range(ndj):
        for zl in range(zblk):
            j = 2 * zl + dj
            jw, jp = j // 2, j % 2
            if jw < win:
                m[dj, jw, jp, zl] = 1.0
    return m


_M1I = _band_mask(5, 28, 12)
_M2I = _band_mask(5, 12, 4)
_M1A = _band_mask(3, 64, 31)
_M2A = _band_mask_s2(3, 7, 6)


def _fused_body(img_ref, aud_ref,
                b1i_ref, ib1_ref, is1_ref, it1_ref,
                b2i_ref, ib2_ref, is2_ref, it2_ref,
                ifw_ref, ifb_ref,
                b1a_ref, ab1_ref,
                b2a_ref, ab2_ref,
                afw_ref, afb_ref,
                w1i_ref, w1a_ref, fb1_ref, w2f_ref, fb2_ref,
                out_ref):
    tb = img_ref.shape[1]
    f32 = jnp.float32
    bf16 = jnp.bfloat16

    x = img_ref[...].astype(bf16)
    xc = jnp.concatenate([x[di:di + 24] for di in range(5)], axis=-1)
    h = jnp.dot(xc.reshape(24 * tb, 140), b1i_ref[...],
                preferred_element_type=f32)
    h = h.reshape(12, 2, tb, 768)
    h = jnp.max(h, axis=1)
    h = jnp.maximum(h[:, :, :384], h[:, :, 384:])
    h = jnp.maximum(h + ib1_ref[...].reshape(1, 1, 384), 0.0)
    h = h * is1_ref[...].reshape(1, 1, 384) + it1_ref[...].reshape(1, 1, 384)

    x2 = h.astype(bf16)
    x2c = jnp.concatenate([x2[di:di + 8] for di in range(5)], axis=-1)
    h2 = jnp.dot(x2c.reshape(8 * tb, 1920), b2i_ref[...],
                 preferred_element_type=f32)
    h2 = h2.reshape(4, 2, tb, 512)
    h2 = jnp.max(h2, axis=1)
    h2 = jnp.maximum(h2[:, :, :256], h2[:, :, 256:])
    h2 = jnp.maximum(h2 + ib2_ref[...].reshape(1, 1, 256), 0.0)
    h2 = h2 * is2_ref[...].reshape(1, 1, 256) + it2_ref[...].reshape(1, 1, 256)
    feat = h2.astype(bf16)

    img_repr = ifb_ref[...]
    for y in range(4):
        img_repr = img_repr + jnp.dot(feat[y], ifw_ref[y],
                                      preferred_element_type=f32)

    a = aud_ref[...].astype(bf16)
    ac = jnp.concatenate([a[di:di + 26] for di in range(3)], axis=-1)
    ha = jnp.dot(ac.reshape(26 * tb, 192), b1a_ref[...],
                 preferred_element_type=f32)
    ha = jnp.maximum(ha + ab1_ref[...], 0.0)
    a1 = ha.astype(bf16).reshape(13, 2, tb, 1984)

    ev, od = a1[:, 0], a1[:, 1]
    blocks = []
    for z0 in (0, 6, 12, 18, 24):
        sl = slice(z0 * 64, z0 * 64 + 448)
        acat = jnp.concatenate(
            [ev[0:12, :, sl], od[0:12, :, sl], ev[1:13, :, sl]], axis=-1)
        blocks.append(jnp.dot(acat.reshape(12 * tb, 1344), b2a_ref[...],
                              preferred_element_type=f32))
    h2a = jnp.concatenate(blocks, axis=-1)
    h2a = jnp.maximum(h2a + ab2_ref[...], 0.0)
    a2 = h2a.astype(bf16).reshape(12, tb, 960)

    aud_repr = afb_ref[...]
    for y in range(12):
        aud_repr = aud_repr + jnp.dot(a2[y], afw_ref[y],
                                      preferred_element_type=f32)

    hf = (jnp.dot(jnp.maximum(img_repr, 0.0), w1i_ref[...],
                  preferred_element_type=f32)
          + jnp.dot(jnp.maximum(aud_repr, 0.0), w1a_ref[...],
                    preferred_element_type=f32)
          + fb1_ref[...])
    hf = jnp.maximum(hf, 0.0)
    out = jnp.dot(hf, w2f_ref[...], preferred_element_type=f32) + fb2_ref[...]
    out_ref[...] = out.astype(out_ref.dtype)


def kernel(image, audio, img_w1_hwio, img_b1, img_w1, img_bn1_s, img_bn1_t,
           img_w2_hwio, img_b2, img_w2, img_bn2_s, img_bn2_t, img_fc_w,
           img_fc_b, aud_w1_hwio, aud_b1, aud_w1, aud_w2_hwio, aud_b2,
           aud_w2, aud_fc_w, aud_fc_b, fus_w1, fus_b1, fus_w1_img,
           fus_w1_aud, fus_w2, fus_b2):
    b = image.shape[0]
    tb = _TB if b % _TB == 0 else (8 if b % 8 == 0 else 1)
    bf16 = jnp.bfloat16

    img_t = image.reshape(b, 28, 28).transpose(1, 0, 2)
    aud_t = audio.reshape(b, 28, 64).transpose(1, 0, 2)

    w1s = img_w1_hwio.reshape(5, 5, 32)
    b1i = jnp.einsum('awpj,dac->dwpjc', _M1I, w1s
                     ).reshape(5 * 28, 768).astype(bf16)
    b2i = jnp.einsum('awpj,daio->dwipjo', _M2I, img_w2_hwio
                     ).reshape(5 * 384, 512).astype(bf16)
    aw1s = aud_w1_hwio.reshape(3, 3, 32)
    b1a = jnp.einsum('awpj,dac->dwjpc', _M1A, aw1s
                     ).reshape(3 * 64, 1984).astype(bf16)
    b2a = jnp.einsum('ajpz,daio->djpizo', _M2A, aud_w2_hwio
                     ).reshape(3 * 448, 192).astype(bf16)
    ab1 = jnp.tile(aud_b1, (1, 62))
    ab2 = jnp.tile(aud_b2, (1, 30))

    ib1 = jnp.tile(img_b1, (1, 12))
    is1 = jnp.tile(img_bn1_s, (1, 12))
    it1 = jnp.tile(img_bn1_t, (1, 12))
    ib2 = jnp.tile(img_b2, (1, 4))
    is2 = jnp.tile(img_bn2_s, (1, 4))
    it2 = jnp.tile(img_bn2_t, (1, 4))
    ifw = img_fc_w.reshape(4, 256, 64).astype(bf16)
    afw = aud_fc_w.reshape(12, 960, 64).astype(bf16)

    def full(a):
        return pl.BlockSpec(a.shape, lambda i: (0,) * a.ndim)

    out = pl.pallas_call(
        _fused_body,
        out_shape=jax.ShapeDtypeStruct((b, 64), jnp.float32),
        grid=(b // tb,),
        in_specs=[
            pl.BlockSpec((28, tb, 28), lambda i: (0, i, 0)),
            pl.BlockSpec((28, tb, 64), lambda i: (0, i, 0)),
            full(b1i), full(ib1), full(is1), full(it1),
            full(b2i), full(ib2), full(is2), full(it2),
            full(ifw), full(img_fc_b),
            full(b1a), full(ab1),
            full(b2a), full(ab2),
            full(afw), full(aud_fc_b),
            full(fus_w1_img), full(fus_w1_aud), full(fus_b1),
            full(fus_w2), full(fus_b2),
        ],
        out_specs=pl.BlockSpec((tb, 64), lambda i: (i, 0)),
        compiler_params=pltpu.CompilerParams(dimension_semantics=("parallel",)),
        cost_estimate=pl.CostEstimate(flops=2 * b * 16_000_000,
                                      transcendentals=0,
                                      bytes_accessed=4 * b * (784 + 1792 + 64)),
    )(img_t, aud_t,
      b1i, ib1, is1, it1,
      b2i, ib2, is2, it2,
      ifw, img_fc_b,
      b1a, ab1,
      b2a, ab2,
      afw, aud_fc_b,
      fus_w1_img, fus_w1_aud, fus_b1, fus_w2, fus_b2)
    return out
